# Initial kernel scaffold; baseline (speedup 1.0000x reference)
import jax
import jax.numpy as jnp
from jax.experimental import pallas as pl


def kernel(y_pred, word_table):
    raise NotImplementedError("write your pallas kernel here")


# speedup vs baseline: 26.4730x; 26.4730x over previous
import functools

import jax
import jax.numpy as jnp
from jax import lax
from jax.experimental import pallas as pl
from jax.experimental.pallas import tpu as pltpu
from jax.experimental.pallas import tpu_sc as plsc

NUM_ROWS = 128
ROW_LEN = 100000
LANES = 16
NUM_CORES = 2
NUM_SUBCORES = 16
NUM_WORKERS = NUM_CORES * NUM_SUBCORES
ROWS_PER_WORKER = NUM_ROWS // NUM_WORKERS
NUM_SLICES = ROW_LEN // LANES
UNROLL = 5
NUM_ITERS = NUM_SLICES // UNROLL

_NEG_INF = float("-inf")
_BIG_IDX = 2**31 - 1


def _combine(m_a, i_a, m_b, i_b):
    take_b = jnp.logical_or(m_b > m_a,
                            jnp.logical_and(m_b == m_a, i_b < i_a))
    return jnp.where(take_b, m_b, m_a), jnp.where(take_b, i_b, i_a)


def _row_argmax(row_ref):
    iota = lax.broadcasted_iota(jnp.int32, (LANES,), 0)

    def body(i, carry):
        ms, mis = carry
        ms, mis = list(ms), list(mis)
        for k in range(UNROLL):
            base = (i * UNROLL + k) * LANES
            v = row_ref[pl.ds(base, LANES)]
            idxv = base + iota
            cmp = v > ms[k]
            ms[k] = jnp.where(cmp, v, ms[k])
            mis[k] = jnp.where(cmp, idxv, mis[k])
        return tuple(ms), tuple(mis)

    init_m = tuple(jnp.full((LANES,), _NEG_INF) for _ in range(UNROLL))
    init_i = tuple(jnp.zeros((LANES,), jnp.int32) for _ in range(UNROLL))
    ms, mis = lax.fori_loop(0, NUM_ITERS, body, (init_m, init_i))

    m, mi = ms[0], mis[0]
    for k in range(1, UNROLL):
        m, mi = _combine(m, mi, ms[k], mis[k])

    bm = m[0]
    bi = mi[0]
    for l in range(1, LANES):
        v = m[l]
        i = mi[l]
        take = jnp.logical_or(v > bm, jnp.logical_and(v == bm, i < bi))
        bm = jnp.where(take, v, bm)
        bi = jnp.where(take, i, bi)
    return bi


def _build_sc_call():
    mesh = plsc.VectorSubcoreMesh(core_axis_name="c", subcore_axis_name="s",
                                  num_cores=NUM_CORES,
                                  num_subcores=NUM_SUBCORES)

    @functools.partial(
        pl.kernel,
        out_type=jax.ShapeDtypeStruct((NUM_WORKERS, LANES), jnp.int32),
        mesh=mesh,
        scratch_types=[
            pltpu.VMEM((ROW_LEN,), jnp.float32),
            pltpu.VMEM((LANES,), jnp.int32),
            pltpu.VMEM((LANES,), jnp.int32),
            pltpu.SemaphoreType.DMA,
        ],
    )
    def sc_kernel(y_hbm, table_hbm, out_hbm, row_buf, idx_buf, word_buf, sem):
        wid = lax.axis_index("s") * NUM_CORES + lax.axis_index("c")
        base_row = wid * ROWS_PER_WORKER
        iota = lax.broadcasted_iota(jnp.int32, (LANES,), 0)

        res_vec = jnp.zeros((LANES,), jnp.int32)
        for r in range(ROWS_PER_WORKER):
            pltpu.sync_copy(y_hbm.at[base_row + r], row_buf)
            a = _row_argmax(row_buf)
            res_vec = jnp.where(iota == r, a, res_vec)

        idx_buf[...] = res_vec
        pltpu.async_copy(table_hbm.at[idx_buf], word_buf, sem).wait()
        pltpu.sync_copy(word_buf, out_hbm.at[wid])

    return sc_kernel


_sc_call = _build_sc_call()


@jax.jit
def kernel(y_pred, word_table):
    staged = _sc_call(y_pred, word_table)
    return staged[:, :ROWS_PER_WORKER].reshape(-1)
